# baseline (speedup 1.0000x reference)
import functools

import jax
import jax.numpy as jnp
from jax import lax
from jax.experimental import pallas as pl
from jax.experimental.pallas import tpu as pltpu
from jax.experimental.pallas import tpu_sc as plsc

MEM_ROWS = 65536
BATCH = 4096
FEAT_DIM = 256
NUM_CORES = 2
NUM_SUBCORES = 16
NUM_WORKERS = NUM_CORES * NUM_SUBCORES

SC_ROWS = BATCH
ROWS_PER_W = SC_ROWS // NUM_WORKERS
SC_CHUNK = 128
SC_NCHUNK = ROWS_PER_W // SC_CHUNK
SC_NBUF = 3
SC_AHEAD = 1

CHUNK = 2048
NCHUNK = MEM_ROWS // CHUNK
XCHUNK = BATCH // CHUNK
NBUF = 8
AHEAD = 4


def _sc_copy(x):
    mesh = plsc.VectorSubcoreMesh(
        core_axis_name="core", subcore_axis_name="subcore"
    )

    @functools.partial(
        pl.kernel,
        out_type=jax.ShapeDtypeStruct((SC_ROWS, FEAT_DIM), jnp.float32),
        mesh=mesh,
        scratch_types=[
            pltpu.VMEM((SC_NBUF, SC_CHUNK, FEAT_DIM), jnp.float32),
            pltpu.SemaphoreType.DMA((SC_NBUF,)),
            pltpu.SemaphoreType.DMA((SC_NBUF,)),
        ],
    )
    def bank(x_hbm, o_hbm, buf, gsem, ssem):
        wid = lax.axis_index("subcore") * NUM_CORES + lax.axis_index("core")
        base = wid * ROWS_PER_W
        gathers, scatters = [None] * SC_NCHUNK, [None] * SC_NCHUNK

        def fire_gather(i):
            b = i % SC_NBUF
            if i >= SC_NBUF:
                scatters[i - SC_NBUF].wait()
            gathers[i] = pltpu.make_async_copy(
                x_hbm.at[pl.ds(base + i * SC_CHUNK, SC_CHUNK)],
                buf.at[b], gsem.at[b])
            gathers[i].start()

        for i in range(SC_AHEAD):
            fire_gather(i)
        for i in range(SC_NCHUNK):
            if i + SC_AHEAD < SC_NCHUNK:
                fire_gather(i + SC_AHEAD)
            b = i % SC_NBUF
            gathers[i].wait()
            scatters[i] = pltpu.make_async_copy(
                buf.at[b],
                o_hbm.at[pl.ds(base + i * SC_CHUNK, SC_CHUNK)], ssem.at[b])
            scatters[i].start()
        for i in range(max(0, SC_NCHUNK - SC_NBUF), SC_NCHUNK):
            scatters[i].wait()

    return bank(x)


def _dma_body(w_ref, f_ref, o_ref, buf, gsem, ssem):
    gathers, scatters = [None] * NCHUNK, [None] * NCHUNK

    def fire_gather(i):
        b = i % NBUF
        if i >= NBUF:
            scatters[i - NBUF].wait()
        src = w_ref if i < XCHUNK else f_ref
        gathers[i] = pltpu.make_async_copy(
            src.at[pl.ds(i * CHUNK, CHUNK)], buf.at[b], gsem.at[b])
        gathers[i].start()

    for i in range(AHEAD):
        fire_gather(i)
    for i in range(NCHUNK):
        if i + AHEAD < NCHUNK:
            fire_gather(i + AHEAD)
        b = i % NBUF
        gathers[i].wait()
        scatters[i] = pltpu.make_async_copy(
            buf.at[b], o_ref.at[pl.ds(i * CHUNK, CHUNK)], ssem.at[b])
        scatters[i].start()
    for i in range(NCHUNK - NBUF, NCHUNK):
        scatters[i].wait()


def _tc_ring(window, feats):
    return pl.pallas_call(
        _dma_body,
        in_specs=[
            pl.BlockSpec(memory_space=pl.ANY),
            pl.BlockSpec(memory_space=pl.ANY),
        ],
        out_specs=pl.BlockSpec(memory_space=pl.ANY),
        out_shape=jax.ShapeDtypeStruct((MEM_ROWS, FEAT_DIM), jnp.float32),
        scratch_shapes=[
            pltpu.VMEM((NBUF, CHUNK, FEAT_DIM), jnp.float32),
            pltpu.SemaphoreType.DMA((NBUF,)),
            pltpu.SemaphoreType.DMA((NBUF,)),
        ],
    )(window, feats)


def kernel(x, feats):
    window = _sc_copy(x)
    return _tc_ring(window, feats)


# speedup vs baseline: 1.0355x; 1.0355x over previous
import functools

import jax
import jax.numpy as jnp
from jax import lax
from jax.experimental import pallas as pl
from jax.experimental.pallas import tpu as pltpu
from jax.experimental.pallas import tpu_sc as plsc

MEM_ROWS = 65536
BATCH = 4096
FEAT_DIM = 256
NUM_CORES = 2
NUM_SUBCORES = 16
NUM_WORKERS = NUM_CORES * NUM_SUBCORES

SPLIT = 57344
SC_ROWS = MEM_ROWS - SPLIT
ROWS_PER_W = SC_ROWS // NUM_WORKERS
SC_CHUNK = 128
SC_NCHUNK = ROWS_PER_W // SC_CHUNK
SC_NBUF = 3
SC_AHEAD = 1

CHUNK = 2048
NCHUNK = SPLIT // CHUNK
XCHUNK = BATCH // CHUNK
NBUF = 8
AHEAD = 4


def _sc_copy(feats):
    mesh = plsc.VectorSubcoreMesh(
        core_axis_name="core", subcore_axis_name="subcore"
    )

    @functools.partial(
        pl.kernel,
        out_type=jax.ShapeDtypeStruct((MEM_ROWS, FEAT_DIM), jnp.float32),
        mesh=mesh,
        scratch_types=[
            pltpu.VMEM((SC_NBUF, SC_CHUNK, FEAT_DIM), jnp.float32),
            pltpu.SemaphoreType.DMA((SC_NBUF,)),
            pltpu.SemaphoreType.DMA((SC_NBUF,)),
        ],
    )
    def bank(f_hbm, o_hbm, buf, gsem, ssem):
        wid = lax.axis_index("subcore") * NUM_CORES + lax.axis_index("core")
        base = wid * ROWS_PER_W
        gathers, scatters = [None] * SC_NCHUNK, [None] * SC_NCHUNK

        def fire_gather(i):
            b = i % SC_NBUF
            if i >= SC_NBUF:
                scatters[i - SC_NBUF].wait()
            gathers[i] = pltpu.make_async_copy(
                f_hbm.at[pl.ds(SPLIT + base + i * SC_CHUNK, SC_CHUNK)],
                buf.at[b], gsem.at[b])
            gathers[i].start()

        for i in range(SC_AHEAD):
            fire_gather(i)
        for i in range(SC_NCHUNK):
            if i + SC_AHEAD < SC_NCHUNK:
                fire_gather(i + SC_AHEAD)
            b = i % SC_NBUF
            gathers[i].wait()
            scatters[i] = pltpu.make_async_copy(
                buf.at[b],
                o_hbm.at[pl.ds(SPLIT + base + i * SC_CHUNK, SC_CHUNK)], ssem.at[b])
            scatters[i].start()
        for i in range(max(0, SC_NCHUNK - SC_NBUF), SC_NCHUNK):
            scatters[i].wait()

    return bank(feats)


def _dma_body(x_ref, f_ref, bank_ref, o_ref, buf, gsem, ssem):
    gathers, scatters = [None] * NCHUNK, [None] * NCHUNK

    def fire_gather(i):
        b = i % NBUF
        if i >= NBUF:
            scatters[i - NBUF].wait()
        src = x_ref if i < XCHUNK else f_ref
        gathers[i] = pltpu.make_async_copy(
            src.at[pl.ds(i * CHUNK, CHUNK)], buf.at[b], gsem.at[b])
        gathers[i].start()

    for i in range(AHEAD):
        fire_gather(i)
    for i in range(NCHUNK):
        if i + AHEAD < NCHUNK:
            fire_gather(i + AHEAD)
        b = i % NBUF
        gathers[i].wait()
        scatters[i] = pltpu.make_async_copy(
            buf.at[b], o_ref.at[pl.ds(i * CHUNK, CHUNK)], ssem.at[b])
        scatters[i].start()
    for i in range(NCHUNK - NBUF, NCHUNK):
        scatters[i].wait()


def _tc_ring(x, feats, bank):
    return pl.pallas_call(
        _dma_body,
        in_specs=[
            pl.BlockSpec(memory_space=pl.ANY),
            pl.BlockSpec(memory_space=pl.ANY),
            pl.BlockSpec(memory_space=pl.ANY),
        ],
        out_specs=pl.BlockSpec(memory_space=pl.ANY),
        out_shape=jax.ShapeDtypeStruct((MEM_ROWS, FEAT_DIM), jnp.float32),
        input_output_aliases={2: 0},
        scratch_shapes=[
            pltpu.VMEM((NBUF, CHUNK, FEAT_DIM), jnp.float32),
            pltpu.SemaphoreType.DMA((NBUF,)),
            pltpu.SemaphoreType.DMA((NBUF,)),
        ],
    )(x, feats, bank)


def kernel(x, feats):
    bank = _sc_copy(feats)
    return _tc_ring(x, feats, bank)
